# baseline (speedup 1.0000x reference)
import functools

import jax
import jax.numpy as jnp
from jax import lax
from jax.experimental import pallas as pl
from jax.experimental.pallas import tpu as pltpu
from jax.experimental.pallas import tpu_sc as plsc

_HID = 64
_SCALE = 8.0

_NC = 2
_NS = 16
_NW = _NC * _NS
_LANES = 16

_CHUNK = 128
_NBUF = 10
_LOOKAHEAD = 5


def _make_kernel(nrows, ncols):
    nchunk = ncols
    nround = nchunk // _NBUF
    assert nround * _NBUF == nchunk
    mesh = plsc.VectorSubcoreMesh(
        core_axis_name="c", subcore_axis_name="s",
        num_cores=_NC, num_subcores=_NS,
    )

    @functools.partial(
        pl.kernel,
        out_type=jax.ShapeDtypeStruct((nrows, ncols, _HID), jnp.float32),
        mesh=mesh,
        scratch_types=(
            [pltpu.VMEM((nchunk, _CHUNK), jnp.int32)]
            + [pltpu.VMEM((_CHUNK, _HID), jnp.float32) for _ in range(_NBUF)]
            + [pltpu.SemaphoreType.DMA for _ in range(2 * _NBUF)]
        ),
        compiler_params=pltpu.CompilerParams(use_tc_tiling_on_sc=False),
    )
    def emb_kernel(table_hbm, idxt_hbm, out_hbm, idx_v, *scratch):
        rows = scratch[:_NBUF]
        gsem = scratch[_NBUF:2 * _NBUF]
        ssem = scratch[2 * _NBUF:]
        wid = lax.axis_index("s") * _NC + lax.axis_index("c")
        i0 = wid * _CHUNK
        pltpu.sync_copy(idxt_hbm.at[:, pl.ds(i0, _CHUNK)], idx_v)

        for b in range(_LOOKAHEAD):
            pltpu.async_copy(table_hbm.at[idx_v.at[b]], rows[b], gsem[b])

        def _scale(buf):
            @pl.loop(0, _CHUNK)
            def _rows(r):
                for c in range(_HID // _LANES):
                    sl = pl.ds(c * _LANES, _LANES)
                    buf[r, sl] = buf[r, sl] * _SCALE

        @pl.loop(0, nround)
        def _round(t):
            for b in range(_NBUF):
                g = t * _NBUF + b
                pb = (b + _LOOKAHEAD) % _NBUF
                pltpu.make_async_copy(
                    table_hbm.at[idx_v.at[g]], rows[b], gsem[b]).wait()

                def _prefetch(t=t, b=b, pb=pb):
                    gf = t * _NBUF + b + _LOOKAHEAD
                    gp = gf - _NBUF
                    pltpu.make_async_copy(
                        rows[pb],
                        out_hbm.at[pl.ds(i0, _CHUNK), gp],
                        ssem[pb]).wait()
                    pltpu.async_copy(
                        table_hbm.at[idx_v.at[gf]], rows[pb], gsem[pb])

                if b < _LOOKAHEAD:
                    @pl.when(t >= 1)
                    def _(t=t, b=b, pb=pb):
                        _prefetch(t, b, pb)

                    @pl.when(t == 0)
                    def _(t=t, b=b, pb=pb):
                        gf = t * _NBUF + b + _LOOKAHEAD
                        pltpu.async_copy(
                            table_hbm.at[idx_v.at[gf]], rows[pb], gsem[pb])
                else:
                    @pl.when(t < nround - 1)
                    def _(t=t, b=b, pb=pb):
                        _prefetch(t, b, pb)

                _scale(rows[b])
                pltpu.async_copy(
                    rows[b], out_hbm.at[pl.ds(i0, _CHUNK), g], ssem[b])

        for b in range(_NBUF):
            g = (nround - 1) * _NBUF + b
            pltpu.make_async_copy(
                rows[b], out_hbm.at[pl.ds(i0, _CHUNK), g], ssem[b]).wait()

    return emb_kernel


def kernel(x, emb_table):
    nrows, ncols = x.shape
    assert nrows == _NW * _CHUNK
    xt = jnp.swapaxes(x.astype(jnp.int32), 0, 1)
    return _make_kernel(nrows, ncols)(emb_table, xt)
